# baseline (speedup 1.0000x reference)
import jax
import jax.numpy as jnp
from jax.experimental import pallas as pl
from jax.experimental.pallas import tpu as pltpu

B = 4; P = 512; D = 1024; H = 16; E = 8; K = 2; DH = 2048; OUT = 2; M = 2
N = M * P
T = B * N
DHD = D // H
SCALE = DHD ** -0.5
RB = 512
NRB = T // RB
EPS = 1e-5
F32 = jnp.float32


def _ln(x, s, b):
    m = jnp.mean(x, axis=-1, keepdims=True)
    v = jnp.mean((x - m) ** 2, axis=-1, keepdims=True)
    return (x - m) * jax.lax.rsqrt(v + EPS) * s + b


def _dot(a, b):
    return jnp.dot(a, b, preferred_element_type=F32,
                   precision=jax.lax.Precision.HIGHEST)


def _gelu(x):
    return 0.5 * x * (1.0 + jax.lax.erf(x * (2.0 ** -0.5)))


def _qkv_body(*args):
    *ins, q_ref, kv_ref = args
    part_refs = ins[:-4]
    n1s, n1b, wq, wkv = ins[-4:]
    x = None
    for r in part_refs:
        v = r[0] if len(r.shape) == 3 else r[...]
        x = v if x is None else x + v
    xn = _ln(x, n1s[...], n1b[...])
    q_ref[...] = _dot(xn, wq[...])
    kv_ref[...] = _dot(xn, wkv[...])


def _ln_qkv(parts, part_specs, n1s, n1b, wq, wkv):
    row_spec = pl.BlockSpec((RB, D), lambda rb: (rb, 0))
    vec_spec = pl.BlockSpec((1, D), lambda rb: (0, 0))
    return pl.pallas_call(
        _qkv_body,
        grid=(NRB,),
        in_specs=[*part_specs, vec_spec, vec_spec,
                  pl.BlockSpec((D, D), lambda rb: (0, 0)),
                  pl.BlockSpec((D, 2 * D), lambda rb: (0, 0))],
        out_specs=[row_spec, pl.BlockSpec((RB, 2 * D), lambda rb: (rb, 0))],
        out_shape=[jax.ShapeDtypeStruct((T, D), F32),
                   jax.ShapeDtypeStruct((T, 2 * D), F32)],
        compiler_params=pltpu.CompilerParams(
            dimension_semantics=("arbitrary",)),
    )(*parts, n1s.reshape(1, D), n1b.reshape(1, D), wq, wkv)


def _attn_body(q_ref, kv_ref, o_ref):
    q = q_ref[0]
    kv = kv_ref[0]
    outs = []
    for h in range(H):
        qh = q[:, h * DHD:(h + 1) * DHD]
        kh = kv[:, h * DHD:(h + 1) * DHD]
        vh = kv[:, D + h * DHD:D + (h + 1) * DHD]
        s = jax.lax.dot_general(qh, kh, (((1,), (1,)), ((), ())),
                                preferred_element_type=F32,
                                precision=jax.lax.Precision.HIGHEST) * SCALE
        mx = jnp.max(s, axis=-1, keepdims=True)
        p = jnp.exp(s - mx)
        p = p / jnp.sum(p, axis=-1, keepdims=True)
        outs.append(_dot(p, vh))
    o_ref[0] = jnp.concatenate(outs, axis=1)


def _attn(q, kv):
    q3 = q.reshape(B, N, D)
    kv3 = kv.reshape(B, N, 2 * D)
    o = pl.pallas_call(
        _attn_body,
        grid=(B,),
        in_specs=[pl.BlockSpec((1, N, D), lambda b: (b, 0, 0)),
                  pl.BlockSpec((1, N, 2 * D), lambda b: (b, 0, 0))],
        out_specs=pl.BlockSpec((1, N, D), lambda b: (b, 0, 0)),
        out_shape=jax.ShapeDtypeStruct((B, N, D), F32),
        compiler_params=pltpu.CompilerParams(
            dimension_semantics=("arbitrary",)),
    )(q3, kv3)
    return o.reshape(T, D)


def _proj_body_router(o_ref, wproj, bproj, n2s, n2b, wg,
                      a2_ref, xn2_ref, g1_ref, g2_ref, i1_ref, i2_ref):
    a = _dot(o_ref[...], wproj[...]) + bproj[...]
    a2 = a + a
    a2_ref[...] = a2
    xn2 = _ln(a2, n2s[...], n2b[...])
    xn2_ref[...] = xn2
    logits = _dot(xn2, wg[...])
    iota = jax.lax.broadcasted_iota(jnp.int32, (RB, E), 1)
    m1 = jnp.max(logits, axis=1, keepdims=True)
    i1 = jnp.min(jnp.where(logits == m1, iota, E), axis=1, keepdims=True)
    l2 = jnp.where(iota == i1, -jnp.inf, logits)
    m2 = jnp.max(l2, axis=1, keepdims=True)
    i2 = jnp.min(jnp.where(l2 == m2, iota, E), axis=1, keepdims=True)
    e2 = jnp.exp(m2 - m1)
    g1 = 1.0 / (1.0 + e2)
    g1_ref[...] = g1
    g2_ref[...] = 1.0 - g1
    i1_ref[...] = i1
    i2_ref[...] = i2


def _proj_body(o_ref, wproj, bproj, n2s, n2b, a2_ref, xn2_ref):
    a = _dot(o_ref[...], wproj[...]) + bproj[...]
    a2 = a + a
    a2_ref[...] = a2
    xn2_ref[...] = _ln(a2, n2s[...], n2b[...])


def _proj(o, wproj, bproj, n2s, n2b, wg=None):
    row_spec = pl.BlockSpec((RB, D), lambda rb: (rb, 0))
    vec_spec = pl.BlockSpec((1, D), lambda rb: (0, 0))
    col_spec = pl.BlockSpec((RB, 1), lambda rb: (rb, 0))
    in_specs = [row_spec,
                pl.BlockSpec((D, D), lambda rb: (0, 0)),
                vec_spec, vec_spec, vec_spec]
    args = [o, wproj, bproj.reshape(1, D), n2s.reshape(1, D), n2b.reshape(1, D)]
    if wg is None:
        body = _proj_body
        out_specs = [row_spec, row_spec]
        out_shape = [jax.ShapeDtypeStruct((T, D), F32)] * 2
    else:
        body = _proj_body_router
        in_specs.append(pl.BlockSpec((D, E), lambda rb: (0, 0)))
        args.append(wg)
        out_specs = [row_spec, row_spec, col_spec, col_spec, col_spec, col_spec]
        out_shape = [jax.ShapeDtypeStruct((T, D), F32)] * 2 + \
                    [jax.ShapeDtypeStruct((T, 1), F32)] * 2 + \
                    [jax.ShapeDtypeStruct((T, 1), jnp.int32)] * 2
    return pl.pallas_call(
        body, grid=(NRB,), in_specs=in_specs, out_specs=out_specs,
        out_shape=out_shape,
        compiler_params=pltpu.CompilerParams(
            dimension_semantics=("arbitrary",)),
    )(*args)


def _moe_body(x_ref, w1_ref, b1_ref, w2_ref, b2_ref,
              g1_ref, g2_ref, i1_ref, i2_ref, out_ref):
    e = pl.program_id(0)
    rb = pl.program_id(1)
    x = x_ref[...]
    h = _gelu(_dot(x, w1_ref[0]) + b1_ref[0])
    y = _dot(h, w2_ref[0]) + b2_ref[0]
    wt = (g1_ref[...] * (i1_ref[...] == e).astype(F32)
          + g2_ref[...] * (i2_ref[...] == e).astype(F32))
    contrib = wt * y
    sl = pl.ds(rb * MRB, MRB)

    @pl.when(e == 0)
    def _():
        out_ref[sl, :] = contrib

    @pl.when(e != 0)
    def _():
        out_ref[sl, :] += contrib


MRB = 256


def _moe_dense(xn2, w1e, b1e, w2e, b2e, g1, g2, i1, i2):
    row_spec = pl.BlockSpec((MRB, D), lambda e, rb: (rb, 0))
    col_spec = pl.BlockSpec((MRB, 1), lambda e, rb: (rb, 0))
    return pl.pallas_call(
        _moe_body,
        grid=(E, T // MRB),
        in_specs=[row_spec,
                  pl.BlockSpec((1, D, DH), lambda e, rb: (e, 0, 0)),
                  pl.BlockSpec((1, 1, DH), lambda e, rb: (e, 0, 0)),
                  pl.BlockSpec((1, DH, D), lambda e, rb: (e, 0, 0)),
                  pl.BlockSpec((1, 1, D), lambda e, rb: (e, 0, 0)),
                  col_spec, col_spec, col_spec, col_spec],
        out_specs=pl.BlockSpec((T, D), lambda e, rb: (0, 0)),
        out_shape=jax.ShapeDtypeStruct((T, D), F32),
        compiler_params=pltpu.CompilerParams(
            dimension_semantics=("arbitrary", "arbitrary")),
    )(xn2, w1e, b1e.reshape(E, 1, DH), w2e, b2e.reshape(E, 1, D),
      g1, g2, i1, i2)


NP = K * T
MB = 256
NB = NP // MB
S = NB + E - 1


def _route_metadata(g1, g2, i1, i2):
    pe = jnp.concatenate([i1[:, 0], i2[:, 0]])
    gg = jnp.concatenate([g1[:, 0], g2[:, 0]])
    onehot = (pe[:, None] == jnp.arange(E, dtype=jnp.int32)[None, :])
    ranks = jnp.cumsum(onehot.astype(jnp.int32), axis=0)
    rank = jnp.take_along_axis(ranks, pe[:, None].astype(jnp.int32), 1)[:, 0] - 1
    counts = ranks[-1]
    offs = jnp.concatenate([jnp.zeros((1,), jnp.int32),
                            jnp.cumsum(counts)[:-1].astype(jnp.int32)])
    pos = offs[pe] + rank
    inv = jnp.zeros((NP,), jnp.int32).at[pos].set(
        jnp.arange(NP, dtype=jnp.int32))
    src_tok = inv % T
    g_sorted = gg[inv]
    bgrid = jnp.arange(NB, dtype=jnp.int32)[:, None]
    egrid = jnp.arange(E, dtype=jnp.int32)[None, :]
    lo = jnp.maximum(bgrid * MB, offs[None, :])
    hi = jnp.minimum((bgrid + 1) * MB, (offs + counts)[None, :])
    valid = (hi > lo).reshape(-1)
    slot = jnp.cumsum(valid.astype(jnp.int32)) - 1
    slot = jnp.where(valid, slot, S)
    def compact(vals, fill):
        return jnp.full((S,), fill, jnp.int32).at[slot].set(
            vals.reshape(-1).astype(jnp.int32), mode="drop")
    step_blk = compact(jnp.broadcast_to(bgrid, (NB, E)), NB - 1)
    step_e = compact(jnp.broadcast_to(egrid, (NB, E)), E - 1)
    step_lo = compact(lo - bgrid * MB, 0)
    step_hi = compact(hi - bgrid * MB, 0)
    return pos, src_tok, g_sorted, step_blk, step_e, step_lo, step_hi


def _moe_grouped_body(blk_ref, e_ref, lo_ref, hi_ref,
                      xs_ref, w1_ref, b1_ref, w2_ref, b2_ref, g_ref, out_ref):
    s = pl.program_id(0)
    blk = blk_ref[s]
    prev = jnp.where(s == 0, -1, blk_ref[jnp.maximum(s - 1, 0)])
    first = prev != blk
    h = _gelu(_dot(xs_ref[...], w1_ref[0]) + b1_ref[0])
    y = _dot(h, w2_ref[0]) + b2_ref[0]
    rid = jax.lax.broadcasted_iota(jnp.int32, (MB, 1), 0)
    mask = (rid >= lo_ref[s]) & (rid < hi_ref[s])
    geff = jnp.where(mask, g_ref[0], 0.0)
    contrib = geff * y

    @pl.when(first)
    def _():
        out_ref[...] = contrib

    @pl.when(jnp.logical_not(first))
    def _():
        out_ref[...] += contrib


def _moe_grouped(xs, g_sorted, w1e, b1e, w2e, b2e,
                 step_blk, step_e, step_lo, step_hi):
    grid_spec = pltpu.PrefetchScalarGridSpec(
        num_scalar_prefetch=4,
        grid=(S,),
        in_specs=[
            pl.BlockSpec((MB, D), lambda s, blk, e, lo, hi: (blk[s], 0)),
            pl.BlockSpec((1, D, DH), lambda s, blk, e, lo, hi: (e[s], 0, 0)),
            pl.BlockSpec((1, 1, DH), lambda s, blk, e, lo, hi: (e[s], 0, 0)),
            pl.BlockSpec((1, DH, D), lambda s, blk, e, lo, hi: (e[s], 0, 0)),
            pl.BlockSpec((1, 1, D), lambda s, blk, e, lo, hi: (e[s], 0, 0)),
            pl.BlockSpec((1, MB, 1), lambda s, blk, e, lo, hi: (blk[s], 0, 0)),
        ],
        out_specs=pl.BlockSpec((MB, D), lambda s, blk, e, lo, hi: (blk[s], 0)),
    )
    return pl.pallas_call(
        _moe_grouped_body,
        grid_spec=grid_spec,
        out_shape=jax.ShapeDtypeStruct((NP, D), F32),
        compiler_params=pltpu.CompilerParams(
            dimension_semantics=("arbitrary",)),
    )(step_blk, step_e, step_lo, step_hi,
      xs, w1e, b1e.reshape(E, 1, DH), w2e, b2e.reshape(E, 1, D),
      g_sorted.reshape(NB, MB, 1))


def _moe_routed(xn2, w1e, b1e, w2e, b2e, g1, g2, i1, i2):
    (pos, src_tok, g_sorted, step_blk, step_e, step_lo,
     step_hi) = _route_metadata(g1, g2, i1, i2)
    xs = jnp.take(xn2, src_tok, axis=0)
    ys = _moe_grouped(xs, g_sorted, w1e, b1e, w2e, b2e,
                      step_blk, step_e, step_lo, step_hi)
    return jnp.take(ys, pos[:T], axis=0) + jnp.take(ys, pos[T:], axis=0)


def _ffn_body(xn_ref, a_ref, w1, b1, w2, b2, out_ref):
    h = _gelu(_dot(xn_ref[...], w1[...]) + b1[...])
    out_ref[...] = a_ref[...] + _dot(h, w2[...]) + b2[...]


def _ffn(xn2, a2, w1, b1, w2, b2):
    row_spec = pl.BlockSpec((RB, D), lambda rb: (rb, 0))
    return pl.pallas_call(
        _ffn_body,
        grid=(NRB,),
        in_specs=[row_spec, row_spec,
                  pl.BlockSpec((D, DH), lambda rb: (0, 0)),
                  pl.BlockSpec((1, DH), lambda rb: (0, 0)),
                  pl.BlockSpec((DH, D), lambda rb: (0, 0)),
                  pl.BlockSpec((1, D), lambda rb: (0, 0))],
        out_specs=row_spec,
        out_shape=jax.ShapeDtypeStruct((T, D), F32),
        compiler_params=pltpu.CompilerParams(
            dimension_semantics=("arbitrary",)),
    )(xn2, a2, w1, b1.reshape(1, DH), w2, b2.reshape(1, D))


def _head_body(x_ref, wh1, bh1, wh2, bh2, out_ref):
    rows = []
    for b in range(B):
        m0 = jnp.sum(x_ref[pl.ds(b * N, P), :], axis=0, keepdims=True) * (1.0 / P)
        m1 = jnp.sum(x_ref[pl.ds(b * N + P, P), :], axis=0, keepdims=True) * (1.0 / P)
        rows.append(jnp.concatenate([m0, m1], axis=1))
    pooled = jnp.concatenate(rows, axis=0)
    hh = jnp.maximum(_dot(pooled, wh1[...]) + bh1[...], 0.0)
    out_ref[...] = _dot(hh, wh2[...]) + bh2[...]


def _head(x2, wh1, bh1, wh2, bh2):
    return pl.pallas_call(
        _head_body,
        grid=(1,),
        in_specs=[pl.BlockSpec((T, D), lambda i: (0, 0)),
                  pl.BlockSpec((M * D, D), lambda i: (0, 0)),
                  pl.BlockSpec((1, D), lambda i: (0, 0)),
                  pl.BlockSpec((D, OUT), lambda i: (0, 0)),
                  pl.BlockSpec((1, OUT), lambda i: (0, 0))],
        out_specs=pl.BlockSpec((B, OUT), lambda i: (0, 0)),
        out_shape=jax.ShapeDtypeStruct((B, OUT), F32),
        compiler_params=pltpu.CompilerParams(
            dimension_semantics=("arbitrary",)),
    )(x2, wh1, bh1.reshape(1, D), wh2, bh2.reshape(1, OUT))


def kernel(x0, x1, pos_embed, l0_n1s, l0_n1b, l0_n2s, l0_n2b, l0_wq, l0_wkv,
           l0_wproj, l0_bproj, l0_wg, l0_w1e, l0_b1e, l0_w2e, l0_b2e,
           l1_n1s, l1_n1b, l1_n2s, l1_n2b, l1_wq, l1_wkv, l1_wproj, l1_bproj,
           l1_w1, l1_b1, l1_w2, l1_b2, wh1, bh1, wh2, bh2):
    x = jnp.concatenate([x0, x1], axis=1).reshape(T, D)
    pos = pos_embed.reshape(M, P, D)

    row_spec = pl.BlockSpec((RB, D), lambda rb: (rb, 0))
    pos_spec = pl.BlockSpec((1, P, D), lambda rb: (rb % M, 0, 0))

    q, kv = _ln_qkv([x, pos], [row_spec, pos_spec], l0_n1s, l0_n1b,
                    l0_wq, l0_wkv)
    o = _attn(q, kv)
    a2, xn2, g1, g2, i1, i2 = _proj(o, l0_wproj, l0_bproj, l0_n2s, l0_n2b,
                                    wg=l0_wg)
    moe_out = _moe_routed(xn2, l0_w1e, l0_b1e, l0_w2e, l0_b2e, g1, g2, i1, i2)

    q1, kv1 = _ln_qkv([a2, moe_out], [row_spec, row_spec], l1_n1s, l1_n1b,
                      l1_wq, l1_wkv)
    o1 = _attn(q1, kv1)
    a2_1, xn2_1 = _proj(o1, l1_wproj, l1_bproj, l1_n2s, l1_n2b)
    x2 = _ffn(xn2_1, a2_1, l1_w1, l1_b1, l1_w2, l1_b2)

    return _head(x2, wh1, bh1, wh2, bh2)


# speedup vs baseline: 3.3242x; 3.3242x over previous
import functools

import jax
import jax.numpy as jnp
from jax import lax
from jax.experimental import pallas as pl
from jax.experimental.pallas import tpu as pltpu
from jax.experimental.pallas import tpu_sc as plsc

B = 4; P = 512; D = 1024; H = 16; E = 8; K = 2; DH = 2048; OUT = 2; M = 2
N = M * P
T = B * N
DHD = D // H
SCALE = DHD ** -0.5
RB = 512
NRB = T // RB
EPS = 1e-5
F32 = jnp.float32


def _ln(x, s, b):
    m = jnp.mean(x, axis=-1, keepdims=True)
    v = jnp.mean((x - m) ** 2, axis=-1, keepdims=True)
    return (x - m) * jax.lax.rsqrt(v + EPS) * s + b


def _dot(a, b):
    return jnp.dot(a, b, preferred_element_type=F32)


def _gelu(x):
    return 0.5 * x * (1.0 + jax.lax.erf(x * (2.0 ** -0.5)))


def _qkv_body(*args):
    *ins, q_ref, kv_ref = args
    part_refs = ins[:-4]
    n1s, n1b, wq, wkv = ins[-4:]
    x = None
    for r in part_refs:
        v = r[0] if len(r.shape) == 3 else r[...]
        x = v if x is None else x + v
    xn = _ln(x, n1s[...], n1b[...])
    q_ref[...] = _dot(xn, wq[...])
    kv_ref[...] = _dot(xn, wkv[...])


def _ln_qkv(parts, part_specs, n1s, n1b, wq, wkv):
    row_spec = pl.BlockSpec((RB, D), lambda rb: (rb, 0))
    vec_spec = pl.BlockSpec((1, D), lambda rb: (0, 0))
    return pl.pallas_call(
        _qkv_body,
        grid=(NRB,),
        in_specs=[*part_specs, vec_spec, vec_spec,
                  pl.BlockSpec((D, D), lambda rb: (0, 0)),
                  pl.BlockSpec((D, 2 * D), lambda rb: (0, 0))],
        out_specs=[row_spec, pl.BlockSpec((RB, 2 * D), lambda rb: (rb, 0))],
        out_shape=[jax.ShapeDtypeStruct((T, D), F32),
                   jax.ShapeDtypeStruct((T, 2 * D), F32)],
        compiler_params=pltpu.CompilerParams(
            dimension_semantics=("arbitrary",)),
    )(*parts, n1s.reshape(1, D), n1b.reshape(1, D), wq, wkv)


def _attn_body(q_ref, kv_ref, o_ref):
    q = q_ref[0]
    kv = kv_ref[0]
    outs = []
    for h in range(H):
        qh = q[:, h * DHD:(h + 1) * DHD]
        kh = kv[:, h * DHD:(h + 1) * DHD]
        vh = kv[:, D + h * DHD:D + (h + 1) * DHD]
        s = jax.lax.dot_general(qh, kh, (((1,), (1,)), ((), ())),
                                preferred_element_type=F32) * SCALE
        mx = jnp.max(s, axis=-1, keepdims=True)
        p = jnp.exp(s - mx)
        p = p / jnp.sum(p, axis=-1, keepdims=True)
        outs.append(_dot(p, vh))
    o_ref[0] = jnp.concatenate(outs, axis=1)


def _attn(q, kv):
    q3 = q.reshape(B, N, D)
    kv3 = kv.reshape(B, N, 2 * D)
    o = pl.pallas_call(
        _attn_body,
        grid=(B,),
        in_specs=[pl.BlockSpec((1, N, D), lambda b: (b, 0, 0)),
                  pl.BlockSpec((1, N, 2 * D), lambda b: (b, 0, 0))],
        out_specs=pl.BlockSpec((1, N, D), lambda b: (b, 0, 0)),
        out_shape=jax.ShapeDtypeStruct((B, N, D), F32),
        compiler_params=pltpu.CompilerParams(
            dimension_semantics=("arbitrary",)),
    )(q3, kv3)
    return o.reshape(T, D)


def _proj_body_router(o_ref, wproj, bproj, n2s, n2b, wg,
                      a2_ref, xn2_ref, g1_ref, g2_ref, i1_ref, i2_ref):
    a = _dot(o_ref[...], wproj[...]) + bproj[...]
    a2 = a + a
    a2_ref[...] = a2
    xn2 = _ln(a2, n2s[...], n2b[...])
    xn2_ref[...] = xn2
    logits = _dot(xn2, wg[...])
    iota = jax.lax.broadcasted_iota(jnp.int32, (RB, E), 1)
    m1 = jnp.max(logits, axis=1, keepdims=True)
    i1 = jnp.min(jnp.where(logits == m1, iota, E), axis=1, keepdims=True)
    l2 = jnp.where(iota == i1, -jnp.inf, logits)
    m2 = jnp.max(l2, axis=1, keepdims=True)
    i2 = jnp.min(jnp.where(l2 == m2, iota, E), axis=1, keepdims=True)
    e2 = jnp.exp(m2 - m1)
    g1 = 1.0 / (1.0 + e2)
    g1_ref[...] = g1
    g2_ref[...] = 1.0 - g1
    i1_ref[...] = i1
    i2_ref[...] = i2


def _proj_body(o_ref, wproj, bproj, n2s, n2b, a2_ref, xn2_ref):
    a = _dot(o_ref[...], wproj[...]) + bproj[...]
    a2 = a + a
    a2_ref[...] = a2
    xn2_ref[...] = _ln(a2, n2s[...], n2b[...])


def _proj(o, wproj, bproj, n2s, n2b, wg=None):
    row_spec = pl.BlockSpec((RB, D), lambda rb: (rb, 0))
    vec_spec = pl.BlockSpec((1, D), lambda rb: (0, 0))
    col_spec = pl.BlockSpec((RB, 1), lambda rb: (rb, 0))
    in_specs = [row_spec,
                pl.BlockSpec((D, D), lambda rb: (0, 0)),
                vec_spec, vec_spec, vec_spec]
    args = [o, wproj, bproj.reshape(1, D), n2s.reshape(1, D), n2b.reshape(1, D)]
    if wg is None:
        body = _proj_body
        out_specs = [row_spec, row_spec]
        out_shape = [jax.ShapeDtypeStruct((T, D), F32)] * 2
    else:
        body = _proj_body_router
        in_specs.append(pl.BlockSpec((D, E), lambda rb: (0, 0)))
        args.append(wg)
        out_specs = [row_spec, row_spec, col_spec, col_spec, col_spec, col_spec]
        out_shape = [jax.ShapeDtypeStruct((T, D), F32)] * 2 + \
                    [jax.ShapeDtypeStruct((T, 1), F32)] * 2 + \
                    [jax.ShapeDtypeStruct((T, 1), jnp.int32)] * 2
    return pl.pallas_call(
        body, grid=(NRB,), in_specs=in_specs, out_specs=out_specs,
        out_shape=out_shape,
        compiler_params=pltpu.CompilerParams(
            dimension_semantics=("arbitrary",)),
    )(*args)


def _moe_body(x_ref, w1_ref, b1_ref, w2_ref, b2_ref,
              g1_ref, g2_ref, i1_ref, i2_ref, out_ref):
    e = pl.program_id(0)
    rb = pl.program_id(1)
    x = x_ref[...]
    h = _gelu(_dot(x, w1_ref[0]) + b1_ref[0])
    y = _dot(h, w2_ref[0]) + b2_ref[0]
    wt = (g1_ref[...] * (i1_ref[...] == e).astype(F32)
          + g2_ref[...] * (i2_ref[...] == e).astype(F32))
    contrib = wt * y
    sl = pl.ds(rb * MRB, MRB)

    @pl.when(e == 0)
    def _():
        out_ref[sl, :] = contrib

    @pl.when(e != 0)
    def _():
        out_ref[sl, :] += contrib


MRB = 256


def _moe_dense(xn2, w1e, b1e, w2e, b2e, g1, g2, i1, i2):
    row_spec = pl.BlockSpec((MRB, D), lambda e, rb: (rb, 0))
    col_spec = pl.BlockSpec((MRB, 1), lambda e, rb: (rb, 0))
    return pl.pallas_call(
        _moe_body,
        grid=(E, T // MRB),
        in_specs=[row_spec,
                  pl.BlockSpec((1, D, DH), lambda e, rb: (e, 0, 0)),
                  pl.BlockSpec((1, 1, DH), lambda e, rb: (e, 0, 0)),
                  pl.BlockSpec((1, DH, D), lambda e, rb: (e, 0, 0)),
                  pl.BlockSpec((1, 1, D), lambda e, rb: (e, 0, 0)),
                  col_spec, col_spec, col_spec, col_spec],
        out_specs=pl.BlockSpec((T, D), lambda e, rb: (0, 0)),
        out_shape=jax.ShapeDtypeStruct((T, D), F32),
        compiler_params=pltpu.CompilerParams(
            dimension_semantics=("arbitrary", "arbitrary")),
    )(xn2, w1e, b1e.reshape(E, 1, DH), w2e, b2e.reshape(E, 1, D),
      g1, g2, i1, i2)


NP = K * T
MB = 256
NB = NP // MB
S = NB + E - 1


def _route_metadata(g1, g2, i1, i2):
    pe = jnp.concatenate([i1[:, 0], i2[:, 0]])
    gg = jnp.concatenate([g1[:, 0], g2[:, 0]])
    onehot = (pe[:, None] == jnp.arange(E, dtype=jnp.int32)[None, :])
    ranks = jnp.cumsum(onehot.astype(jnp.int32), axis=0)
    rank = jnp.take_along_axis(ranks, pe[:, None].astype(jnp.int32), 1)[:, 0] - 1
    counts = ranks[-1]
    offs = jnp.concatenate([jnp.zeros((1,), jnp.int32),
                            jnp.cumsum(counts)[:-1].astype(jnp.int32)])
    pos = offs[pe] + rank
    inv = jnp.zeros((NP,), jnp.int32).at[pos].set(
        jnp.arange(NP, dtype=jnp.int32))
    src_tok = inv % T
    g_sorted = gg[inv]
    bgrid = jnp.arange(NB, dtype=jnp.int32)[:, None]
    egrid = jnp.arange(E, dtype=jnp.int32)[None, :]
    lo = jnp.maximum(bgrid * MB, offs[None, :])
    hi = jnp.minimum((bgrid + 1) * MB, (offs + counts)[None, :])
    valid = (hi > lo).reshape(-1)
    slot = jnp.cumsum(valid.astype(jnp.int32)) - 1
    slot = jnp.where(valid, slot, S)
    def compact(vals, fill):
        return jnp.full((S,), fill, jnp.int32).at[slot].set(
            vals.reshape(-1).astype(jnp.int32), mode="drop")
    step_blk = compact(jnp.broadcast_to(bgrid, (NB, E)), NB - 1)
    step_e = compact(jnp.broadcast_to(egrid, (NB, E)), E - 1)
    step_lo = compact(lo - bgrid * MB, 0)
    step_hi = compact(hi - bgrid * MB, 0)
    return pos, src_tok, g_sorted, step_blk, step_e, step_lo, step_hi


def _moe_grouped_body(blk_ref, e_ref, lo_ref, hi_ref,
                      xs_ref, w1_ref, b1_ref, w2_ref, b2_ref, g_ref, out_ref):
    s = pl.program_id(0)
    blk = blk_ref[s]
    prev = jnp.where(s == 0, -1, blk_ref[jnp.maximum(s - 1, 0)])
    first = prev != blk
    h = _gelu(_dot(xs_ref[...], w1_ref[0]) + b1_ref[0])
    y = _dot(h, w2_ref[0]) + b2_ref[0]
    rid = jax.lax.broadcasted_iota(jnp.int32, (MB, 1), 0)
    mask = (rid >= lo_ref[s]) & (rid < hi_ref[s])
    geff = jnp.where(mask, g_ref[0], 0.0)
    contrib = geff * y

    @pl.when(first)
    def _():
        out_ref[...] = contrib

    @pl.when(jnp.logical_not(first))
    def _():
        out_ref[...] += contrib


def _moe_grouped(xs, g_sorted, w1e, b1e, w2e, b2e,
                 step_blk, step_e, step_lo, step_hi):
    grid_spec = pltpu.PrefetchScalarGridSpec(
        num_scalar_prefetch=4,
        grid=(S,),
        in_specs=[
            pl.BlockSpec((MB, D), lambda s, blk, e, lo, hi: (blk[s], 0)),
            pl.BlockSpec((1, D, DH), lambda s, blk, e, lo, hi: (e[s], 0, 0)),
            pl.BlockSpec((1, 1, DH), lambda s, blk, e, lo, hi: (e[s], 0, 0)),
            pl.BlockSpec((1, DH, D), lambda s, blk, e, lo, hi: (e[s], 0, 0)),
            pl.BlockSpec((1, 1, D), lambda s, blk, e, lo, hi: (e[s], 0, 0)),
            pl.BlockSpec((1, MB, 1), lambda s, blk, e, lo, hi: (blk[s], 0, 0)),
        ],
        out_specs=pl.BlockSpec((MB, D), lambda s, blk, e, lo, hi: (blk[s], 0)),
    )
    return pl.pallas_call(
        _moe_grouped_body,
        grid_spec=grid_spec,
        out_shape=jax.ShapeDtypeStruct((NP, D), F32),
        compiler_params=pltpu.CompilerParams(
            dimension_semantics=("arbitrary",)),
    )(step_blk, step_e, step_lo, step_hi,
      xs, w1e, b1e.reshape(E, 1, DH), w2e, b2e.reshape(E, 1, D),
      g_sorted.reshape(NB, MB, 1))


def _sc_gather(table, idx):
    v_rows, d = table.shape
    bn = idx.shape[0]
    info = plsc.get_sparse_core_info()
    nw = info.num_cores * info.num_subcores
    b_per_w = bn // nw
    ch = 64
    nch = b_per_w // ch
    mesh = plsc.VectorSubcoreMesh(core_axis_name="c", subcore_axis_name="s")

    @functools.partial(
        pl.kernel, mesh=mesh,
        out_type=jax.ShapeDtypeStruct((bn, d), F32),
        scratch_types=[
            pltpu.VMEM((ch,), jnp.int32),
            pltpu.VMEM((ch, d), F32),
            pltpu.SemaphoreType.DMA,
        ],
    )
    def k(table_hbm, idx_hbm, out_hbm, idx_v, rows_v, sem):
        wid = lax.axis_index("s") * info.num_cores + lax.axis_index("c")
        base = wid * b_per_w
        for c in range(nch):
            pltpu.sync_copy(idx_hbm.at[pl.ds(base + c * ch, ch)], idx_v)
            pltpu.async_copy(table_hbm.at[idx_v], rows_v, sem).wait()
            pltpu.sync_copy(rows_v, out_hbm.at[pl.ds(base + c * ch, ch)])

    return k(table, idx)


def _moe_routed(xn2, w1e, b1e, w2e, b2e, g1, g2, i1, i2):
    (pos, src_tok, g_sorted, step_blk, step_e, step_lo,
     step_hi) = _route_metadata(g1, g2, i1, i2)
    xs = _sc_gather(xn2, src_tok)
    ys = _moe_grouped(xs, g_sorted, w1e, b1e, w2e, b2e,
                      step_blk, step_e, step_lo, step_hi)
    mm = _sc_gather(ys, pos)
    return mm[:T], mm[T:]


def _ffn_body(xn_ref, a_ref, w1, b1, w2, b2, out_ref):
    h = _gelu(_dot(xn_ref[...], w1[...]) + b1[...])
    out_ref[...] = a_ref[...] + _dot(h, w2[...]) + b2[...]


def _ffn(xn2, a2, w1, b1, w2, b2):
    row_spec = pl.BlockSpec((RB, D), lambda rb: (rb, 0))
    return pl.pallas_call(
        _ffn_body,
        grid=(NRB,),
        in_specs=[row_spec, row_spec,
                  pl.BlockSpec((D, DH), lambda rb: (0, 0)),
                  pl.BlockSpec((1, DH), lambda rb: (0, 0)),
                  pl.BlockSpec((DH, D), lambda rb: (0, 0)),
                  pl.BlockSpec((1, D), lambda rb: (0, 0))],
        out_specs=row_spec,
        out_shape=jax.ShapeDtypeStruct((T, D), F32),
        compiler_params=pltpu.CompilerParams(
            dimension_semantics=("arbitrary",)),
    )(xn2, a2, w1, b1.reshape(1, DH), w2, b2.reshape(1, D))


def _head_body(x_ref, wh1, bh1, wh2, bh2, out_ref):
    rows = []
    for b in range(B):
        m0 = jnp.sum(x_ref[pl.ds(b * N, P), :], axis=0, keepdims=True) * (1.0 / P)
        m1 = jnp.sum(x_ref[pl.ds(b * N + P, P), :], axis=0, keepdims=True) * (1.0 / P)
        rows.append(jnp.concatenate([m0, m1], axis=1))
    pooled = jnp.concatenate(rows, axis=0)
    hh = jnp.maximum(_dot(pooled, wh1[...]) + bh1[...], 0.0)
    out_ref[...] = _dot(hh, wh2[...]) + bh2[...]


def _head(x2, wh1, bh1, wh2, bh2):
    return pl.pallas_call(
        _head_body,
        grid=(1,),
        in_specs=[pl.BlockSpec((T, D), lambda i: (0, 0)),
                  pl.BlockSpec((M * D, D), lambda i: (0, 0)),
                  pl.BlockSpec((1, D), lambda i: (0, 0)),
                  pl.BlockSpec((D, OUT), lambda i: (0, 0)),
                  pl.BlockSpec((1, OUT), lambda i: (0, 0))],
        out_specs=pl.BlockSpec((B, OUT), lambda i: (0, 0)),
        out_shape=jax.ShapeDtypeStruct((B, OUT), F32),
        compiler_params=pltpu.CompilerParams(
            dimension_semantics=("arbitrary",)),
    )(x2, wh1, bh1.reshape(1, D), wh2, bh2.reshape(1, OUT))


def kernel(x0, x1, pos_embed, l0_n1s, l0_n1b, l0_n2s, l0_n2b, l0_wq, l0_wkv,
           l0_wproj, l0_bproj, l0_wg, l0_w1e, l0_b1e, l0_w2e, l0_b2e,
           l1_n1s, l1_n1b, l1_n2s, l1_n2b, l1_wq, l1_wkv, l1_wproj, l1_bproj,
           l1_w1, l1_b1, l1_w2, l1_b2, wh1, bh1, wh2, bh2):
    x = jnp.concatenate([x0, x1], axis=1).reshape(T, D)
    pos = pos_embed.reshape(M, P, D)

    row_spec = pl.BlockSpec((RB, D), lambda rb: (rb, 0))
    pos_spec = pl.BlockSpec((1, P, D), lambda rb: (rb % M, 0, 0))

    q, kv = _ln_qkv([x, pos], [row_spec, pos_spec], l0_n1s, l0_n1b,
                    l0_wq, l0_wkv)
    o = _attn(q, kv)
    a2, xn2, g1, g2, i1, i2 = _proj(o, l0_wproj, l0_bproj, l0_n2s, l0_n2b,
                                    wg=l0_wg)
    m0, m1 = _moe_routed(xn2, l0_w1e, l0_b1e, l0_w2e, l0_b2e, g1, g2, i1, i2)

    q1, kv1 = _ln_qkv([a2, m0, m1], [row_spec, row_spec, row_spec],
                      l1_n1s, l1_n1b, l1_wq, l1_wkv)
    o1 = _attn(q1, kv1)
    a2_1, xn2_1 = _proj(o1, l1_wproj, l1_bproj, l1_n2s, l1_n2b)
    x2 = _ffn(xn2_1, a2_1, l1_w1, l1_b1, l1_w2, l1_b2)

    return _head(x2, wh1, bh1, wh2, bh2)
